# baseline (speedup 1.0000x reference)
import jax
import jax.numpy as jnp
from jax.experimental import pallas as pl
from jax.experimental.pallas import tpu as pltpu

NUM_SELECT = 300


def _matmul_body(l_ref, m_ref, out_ref):
    p = jax.nn.sigmoid(l_ref[...])
    out_ref[...] = jnp.dot(p, m_ref[...], preferred_element_type=jnp.float32)


def _prob_to_label(pred_logits, pos_maps):
    B, Q, T = pred_logits.shape
    C = pos_maps.shape[0]
    CP = 1280
    RB = 128
    rows = B * Q
    l2 = pred_logits.reshape(rows, T)
    mt = jnp.zeros((T, CP), jnp.float32).at[:, :C].set(pos_maps.T)
    grid = (rows // RB,)
    out = pl.pallas_call(
        _matmul_body,
        grid=grid,
        in_specs=[
            pl.BlockSpec((RB, T), lambda i: (i, 0)),
            pl.BlockSpec((T, CP), lambda i: (0, 0)),
        ],
        out_specs=pl.BlockSpec((RB, CP), lambda i: (i, 0)),
        out_shape=jax.ShapeDtypeStruct((rows, CP), jnp.float32),
    )(l2, mt)
    return out.reshape(B, Q, CP)[:, :, :C]


def kernel(pred_logits, pred_boxes, pos_maps, target_sizes):
    B, Q, T = pred_logits.shape
    C = pos_maps.shape[0]
    prob = _prob_to_label(pred_logits, pos_maps)
    flat = prob.reshape(B, Q * C)
    topk_values, topk_indexes = jax.lax.top_k(flat, NUM_SELECT)
    topk_boxes = topk_indexes // C
    labels = topk_indexes % C
    cx, cy, w, h = jnp.split(pred_boxes, 4, axis=-1)
    boxes = jnp.concatenate(
        [cx - 0.5 * w, cy - 0.5 * h, cx + 0.5 * w, cy + 0.5 * h], axis=-1)
    boxes = jnp.take_along_axis(boxes, topk_boxes[..., None], axis=1)
    img_h = target_sizes[:, 0].astype(jnp.float32)
    img_w = target_sizes[:, 1].astype(jnp.float32)
    scale_fct = jnp.stack([img_w, img_h, img_w, img_h], axis=1)
    boxes = boxes * scale_fct[:, None, :]
    return jnp.concatenate(
        [boxes, topk_values[..., None], labels[..., None].astype(jnp.float32)],
        axis=-1)


# speedup vs baseline: 10.4072x; 10.4072x over previous
import functools

import jax
import jax.numpy as jnp
from jax import lax
from jax.experimental import pallas as pl
from jax.experimental.pallas import tpu as pltpu
from jax.experimental.pallas import tpu_sc as plsc

NUM_SELECT = 300
CP = 1280
C_REAL = 1203
Q = 900
QP = 960
B = 32
T = 512
ROWS_PER_WIN = 32
NWIN = QP // ROWS_PER_WIN
CAP = 4096
REBUILD_AT = CAP - CP
NBINS = 1024
LANE = 16
OUTW = 1824


def _matmul_body(l_ref, m_ref, out_ref):
    p = jax.nn.sigmoid(l_ref[...])
    acc = jnp.dot(p, m_ref[...], preferred_element_type=jnp.float32)
    col = lax.broadcasted_iota(jnp.int32, acc.shape, 1)
    row = lax.broadcasted_iota(jnp.int32, acc.shape, 0)
    q = (pl.program_id(0) * acc.shape[0] + row) % QP
    pad = (col >= C_REAL) | (q >= Q)
    out_ref[...] = jnp.where(pad, -1.0, acc)


def _prob_to_label(pred_logits, pos_maps):
    rows = B * QP
    RB = 128
    lp = jnp.pad(pred_logits, ((0, 0), (0, QP - Q), (0, 0)))
    l2 = lp.reshape(rows, T)
    mt = jnp.zeros((T, CP), jnp.float32).at[:, :C_REAL].set(pos_maps.T)
    out = pl.pallas_call(
        _matmul_body,
        grid=(rows // RB,),
        in_specs=[
            pl.BlockSpec((RB, T), lambda i: (i, 0)),
            pl.BlockSpec((T, CP), lambda i: (0, 0)),
        ],
        out_specs=pl.BlockSpec((RB, CP), lambda i: (i, 0)),
        out_shape=jax.ShapeDtypeStruct((rows, CP), jnp.float32),
    )(l2, mt)
    return out.reshape(B, QP, CP)


def _histo_select(cand_val, cand_idx, hist, btot, rcum, n):
    lane = lax.iota(jnp.int32, LANE)
    nv = (n + LANE - 1) // LANE
    inf16 = jnp.full((LANE,), jnp.inf, jnp.float32)
    ninf16 = jnp.full((LANE,), -jnp.inf, jnp.float32)

    def mm_body(i, c):
        mn, mx = c
        v = cand_val[pl.ds(i * LANE, LANE)]
        lm = (i * LANE + lane) < n
        mn = jnp.minimum(mn, jnp.where(lm, v, inf16))
        mx = jnp.maximum(mx, jnp.where(lm, v, ninf16))
        return mn, mx

    mnv, mxv = lax.fori_loop(0, nv, mm_body, (inf16, ninf16))
    lo = -jnp.max(-mnv)
    hi = jnp.max(mxv)
    rng = jnp.maximum(hi - lo, jnp.float32(0.0))
    e = lax.shift_right_logical(lax.bitcast_convert_type(rng, jnp.int32), 23)
    s = lax.bitcast_convert_type(lax.shift_left(253 - e, 23), jnp.float32)
    scale = s * NBINS

    def binof(v):
        b = ((v - lo) * scale).astype(jnp.int32)
        return jnp.clip(b, 0, NBINS - 1)

    zero16 = jnp.zeros((LANE,), jnp.int32)

    def zbody(g, _):
        hist[pl.ds(g * LANE, LANE)] = zero16
        return 0

    lax.fori_loop(0, NBINS, zbody, 0)

    ones16 = jnp.ones((LANE,), jnp.int32)

    def hbody(i, _):
        v = cand_val[pl.ds(i * LANE, LANE)]
        lm = (i * LANE + lane) < n
        b = binof(v)
        plsc.addupdate_scatter(hist, [b * LANE + lane], ones16, mask=lm)
        return 0

    lax.fori_loop(0, nv, hbody, 0)

    def tbody(g, _):
        base = (g * LANE + lane) * LANE
        acc = zero16
        for l in range(LANE):
            acc = acc + plsc.load_gather(hist, [base + l])
        btot[pl.ds(g * LANE, LANE)] = acc
        return 0

    lax.fori_loop(0, NBINS // LANE, tbody, 0)

    def rbody(g, c):
        carry, cnt = c
        gg = (NBINS // LANE - 1) - g
        ch = btot[pl.ds(gg * LANE, LANE)]
        cs = plsc.cumsum(lax.rev(ch, (0,))) + carry
        rc = lax.rev(cs, (0,))
        rcum[pl.ds(gg * LANE, LANE)] = rc
        cnt = cnt + jnp.sum((rc >= NUM_SELECT).astype(jnp.int32))
        return jnp.max(cs), cnt

    _, cnt = lax.fori_loop(0, NBINS // LANE, rbody,
                           (jnp.int32(0), jnp.int32(0)))
    istar = cnt - 1

    def cbody(i, c):
        k, mn = c
        v = cand_val[pl.ds(i * LANE, LANE)]
        ix = cand_idx[pl.ds(i * LANE, LANE)]
        lm = (i * LANE + lane) < n
        keep = lm & (binof(v) >= istar)
        cs = plsc.cumsum(keep.astype(jnp.int32))
        pos = k + cs - 1
        plsc.store_scatter(cand_val, [pos], v, mask=keep)
        plsc.store_scatter(cand_idx, [pos], ix, mask=keep)
        mn = jnp.minimum(mn, jnp.where(keep, v, inf16))
        return k + jnp.max(cs), mn

    k, mnv2 = lax.fori_loop(0, nv, cbody, (jnp.int32(0), inf16))
    return k, -jnp.max(-mnv2)


def _sc_body(p_hbm, boxes_hbm, imgw_hbm, imgh_hbm, out_hbm,
             win, cand_val, cand_idx, hist, btot, rcum,
             sval, sidx, boxbuf, imgw, imgh, outbuf):
    nc = 2
    b = lax.axis_index("s") * nc + lax.axis_index("c")
    lane = lax.iota(jnp.int32, LANE)

    pltpu.sync_copy(boxes_hbm.at[pl.ds(b * Q * 4, Q * 4)], boxbuf)
    pltpu.sync_copy(imgw_hbm.at[pl.ds(b * LANE, LANE)], imgw)
    pltpu.sync_copy(imgh_hbm.at[pl.ds(b * LANE, LANE)], imgh)

    def rebuild(n, _theta):
        return _histo_select(cand_val, cand_idx, hist, btot, rcum, n)

    def wbody(w, c):
        n, theta = c
        pltpu.sync_copy(p_hbm.at[b, pl.ds(w * ROWS_PER_WIN, ROWS_PER_WIN)],
                        win)

        def rowbody(r, c2):
            n2, th2 = c2
            n2, th2 = lax.cond(n2 >= REBUILD_AT, rebuild,
                               lambda a, t: (a, t), n2, th2)
            rowbase = (w * ROWS_PER_WIN + r) * CP

            r16 = jnp.full((LANE,), r, jnp.int32)

            def ccbody(cc, c3):
                n3, th3 = c3
                v = plsc.load_gather(win, [r16, cc * LANE + lane])
                m = v > th3

                def append(nn):
                    gidx = rowbase + cc * LANE + lane
                    cs = plsc.cumsum(m.astype(jnp.int32))
                    pos = nn + cs - 1
                    plsc.store_scatter(cand_val, [pos], v, mask=m)
                    plsc.store_scatter(cand_idx, [pos], gidx, mask=m)
                    return nn + jnp.max(cs)

                n3 = lax.cond(jnp.any(m), append, lambda nn: nn, n3)
                return n3, th3

            return lax.fori_loop(0, CP // LANE, ccbody, (n2, th2))

        return lax.fori_loop(0, ROWS_PER_WIN, rowbody, (n, theta))

    n, _ = lax.fori_loop(0, NWIN, wbody,
                         (jnp.int32(0), jnp.float32(-0.5)))

    m, _ = _histo_select(cand_val, cand_idx, hist, btot, rcum, n)

    nv = (m + LANE - 1) // LANE

    def rankbody(iv, _):
        vi = cand_val[pl.ds(iv * LANE, LANE)]
        ii = cand_idx[pl.ds(iv * LANE, LANE)]
        lm = (iv * LANE + lane) < m

        def jbody(j, r):
            js = jnp.full((LANE,), j, jnp.int32)
            vj = plsc.load_gather(cand_val, [js])
            ij = plsc.load_gather(cand_idx, [js])
            beat = (vj > vi) | ((vj == vi) & (ij < ii))
            return r + beat.astype(jnp.int32)

        rank = lax.fori_loop(0, m, jbody, jnp.zeros((LANE,), jnp.int32))
        ok = lm & (rank < NUM_SELECT)
        plsc.store_scatter(sval, [rank], vi, mask=ok)
        plsc.store_scatter(sidx, [rank], ii, mask=ok)
        return 0

    lax.fori_loop(0, nv, rankbody, 0)

    wv = imgw[...]
    hv = imgh[...]
    inv5 = jnp.float32(0.2)
    for k in range(NUM_SELECT // LANE + 1):
        r = k * LANE + lane
        ok = r < NUM_SELECT
        v = sval[pl.ds(k * LANE, LANE)]
        ix = sidx[pl.ds(k * LANE, LANE)]
        t = lax.shift_right_logical(ix, 8)
        q = (t.astype(jnp.float32) * inv5).astype(jnp.int32)
        label = ix - q * CP
        bq = q * 4
        cx = plsc.load_gather(boxbuf, [bq], mask=ok)
        cy = plsc.load_gather(boxbuf, [bq + 1], mask=ok)
        bw = plsc.load_gather(boxbuf, [bq + 2], mask=ok)
        bh = plsc.load_gather(boxbuf, [bq + 3], mask=ok)
        x0 = (cx - 0.5 * bw) * wv
        y0 = (cy - 0.5 * bh) * hv
        x1 = (cx + 0.5 * bw) * wv
        y1 = (cy + 0.5 * bh) * hv
        base6 = r * 6
        plsc.store_scatter(outbuf, [base6], x0, mask=ok)
        plsc.store_scatter(outbuf, [base6 + 1], y0, mask=ok)
        plsc.store_scatter(outbuf, [base6 + 2], x1, mask=ok)
        plsc.store_scatter(outbuf, [base6 + 3], y1, mask=ok)
        plsc.store_scatter(outbuf, [base6 + 4], v, mask=ok)
        plsc.store_scatter(outbuf, [base6 + 5],
                           label.astype(jnp.float32), mask=ok)

    pltpu.sync_copy(outbuf, out_hbm.at[pl.ds(b * OUTW, OUTW)])


def _sc_select(p3, boxes, imgw, imgh):
    mesh = plsc.VectorSubcoreMesh(core_axis_name="c", subcore_axis_name="s")
    f = pl.kernel(
        _sc_body,
        out_type=jax.ShapeDtypeStruct((B * OUTW,), jnp.float32),
        mesh=mesh,
        compiler_params=pltpu.CompilerParams(needs_layout_passes=False),
        scratch_types=[
            pltpu.VMEM((ROWS_PER_WIN, CP), jnp.float32),
            pltpu.VMEM((CAP,), jnp.float32),
            pltpu.VMEM((CAP,), jnp.int32),
            pltpu.VMEM((NBINS * LANE,), jnp.int32),
            pltpu.VMEM((NBINS,), jnp.int32),
            pltpu.VMEM((NBINS,), jnp.int32),
            pltpu.VMEM((304,), jnp.float32),
            pltpu.VMEM((304,), jnp.int32),
            pltpu.VMEM((Q * 4,), jnp.float32),
            pltpu.VMEM((LANE,), jnp.float32),
            pltpu.VMEM((LANE,), jnp.float32),
            pltpu.VMEM((OUTW,), jnp.float32),
        ],
    )
    return f(p3, boxes, imgw, imgh)


def kernel(pred_logits, pred_boxes, pos_maps, target_sizes):
    p3 = _prob_to_label(pred_logits, pos_maps)
    img_h = target_sizes[:, 0].astype(jnp.float32)
    img_w = target_sizes[:, 1].astype(jnp.float32)
    imgw = jnp.broadcast_to(img_w[:, None], (B, LANE))
    imgh = jnp.broadcast_to(img_h[:, None], (B, LANE))
    out = _sc_select(p3, pred_boxes.reshape(B * Q * 4), imgw.reshape(-1),
                     imgh.reshape(-1))
    return out.reshape(B, OUTW)[:, :NUM_SELECT * 6].reshape(
        B, NUM_SELECT, 6)


# speedup vs baseline: 28.9780x; 2.7844x over previous
import functools

import jax
import jax.numpy as jnp
from jax import lax
from jax.experimental import pallas as pl
from jax.experimental.pallas import tpu as pltpu
from jax.experimental.pallas import tpu_sc as plsc

NUM_SELECT = 300
CP = 1280
C_REAL = 1203
Q = 900
QP = 960
B = 32
T = 512
WIN = 40960
NWIN = QP * CP // WIN
SUB = 256
CAP = 4096
REBUILD_AT = 1024
NBINS = 1024
LANE = 16
OUTW = 1824


def _matmul_body(l_ref, m_ref, out_ref):
    p = jax.nn.sigmoid(l_ref[...])
    acc = jnp.dot(p, m_ref[...], preferred_element_type=jnp.float32)
    col = lax.broadcasted_iota(jnp.int32, acc.shape, 1)
    row = lax.broadcasted_iota(jnp.int32, acc.shape, 0)
    q = (pl.program_id(0) * acc.shape[0] + row) % QP
    pad = (col >= C_REAL) | (q >= Q)
    out_ref[...] = jnp.where(pad, -1.0, acc)


def _prob_to_label(pred_logits, pos_maps):
    rows = B * QP
    RB = 128
    lp = jnp.pad(pred_logits, ((0, 0), (0, QP - Q), (0, 0)))
    l2 = lp.reshape(rows, T)
    mt = jnp.zeros((T, CP), jnp.float32).at[:, :C_REAL].set(pos_maps.T)
    out = pl.pallas_call(
        _matmul_body,
        grid=(rows // RB,),
        in_specs=[
            pl.BlockSpec((RB, T), lambda i: (i, 0)),
            pl.BlockSpec((T, CP), lambda i: (0, 0)),
        ],
        out_specs=pl.BlockSpec((RB, CP), lambda i: (i, 0)),
        out_shape=jax.ShapeDtypeStruct((rows, CP), jnp.float32),
    )(l2, mt)
    return out.reshape(B, QP, CP)


def _histo_select(cand_val, cand_idx, hist, btot, rcum, n):
    lane = lax.iota(jnp.int32, LANE)
    nv = (n + LANE - 1) // LANE
    inf16 = jnp.full((LANE,), jnp.inf, jnp.float32)
    ninf16 = jnp.full((LANE,), -jnp.inf, jnp.float32)

    def mm_body(i, c):
        mn, mx = c
        v = cand_val[pl.ds(i * LANE, LANE)]
        lm = (i * LANE + lane) < n
        mn = jnp.minimum(mn, jnp.where(lm, v, inf16))
        mx = jnp.maximum(mx, jnp.where(lm, v, ninf16))
        return mn, mx

    mnv, mxv = lax.fori_loop(0, nv, mm_body, (inf16, ninf16))
    lo = -jnp.max(-mnv)
    hi = jnp.max(mxv)
    rng = jnp.maximum(hi - lo, jnp.float32(0.0))
    e = lax.shift_right_logical(lax.bitcast_convert_type(rng, jnp.int32), 23)
    s = lax.bitcast_convert_type(lax.shift_left(253 - e, 23), jnp.float32)
    scale = s * NBINS

    def binof(v):
        b = ((v - lo) * scale).astype(jnp.int32)
        return jnp.clip(b, 0, NBINS - 1)

    zero16 = jnp.zeros((LANE,), jnp.int32)

    def zbody(g, _):
        hist[pl.ds(g * LANE, LANE)] = zero16
        return 0

    lax.fori_loop(0, NBINS, zbody, 0)

    ones16 = jnp.ones((LANE,), jnp.int32)

    def hbody(i, _):
        v = cand_val[pl.ds(i * LANE, LANE)]
        lm = (i * LANE + lane) < n
        b = binof(v)
        plsc.addupdate_scatter(hist, [b * LANE + lane], ones16, mask=lm)
        return 0

    lax.fori_loop(0, nv, hbody, 0)

    def tbody(g, _):
        base = (g * LANE + lane) * LANE
        acc = zero16
        for l in range(LANE):
            acc = acc + plsc.load_gather(hist, [base + l])
        btot[pl.ds(g * LANE, LANE)] = acc
        return 0

    lax.fori_loop(0, NBINS // LANE, tbody, 0)

    def rbody(g, c):
        carry, cnt = c
        gg = (NBINS // LANE - 1) - g
        ch = btot[pl.ds(gg * LANE, LANE)]
        cs = plsc.cumsum(lax.rev(ch, (0,))) + carry
        rc = lax.rev(cs, (0,))
        rcum[pl.ds(gg * LANE, LANE)] = rc
        cnt = cnt + jnp.sum((rc >= NUM_SELECT).astype(jnp.int32))
        return jnp.max(cs), cnt

    _, cnt = lax.fori_loop(0, NBINS // LANE, rbody,
                           (jnp.int32(0), jnp.int32(0)))
    istar = cnt - 1

    def cbody(i, c):
        k, mn = c
        v = cand_val[pl.ds(i * LANE, LANE)]
        ix = cand_idx[pl.ds(i * LANE, LANE)]
        lm = (i * LANE + lane) < n
        keep = lm & (binof(v) >= istar)
        cs = plsc.cumsum(keep.astype(jnp.int32))
        pos = k + cs - 1
        plsc.store_scatter(cand_val, [pos], v, mask=keep)
        plsc.store_scatter(cand_idx, [pos], ix, mask=keep)
        mn = jnp.minimum(mn, jnp.where(keep, v, inf16))
        return k + jnp.max(cs), mn

    k, mnv2 = lax.fori_loop(0, nv, cbody, (jnp.int32(0), inf16))
    return k, -jnp.max(-mnv2)


def _sc_body(p_hbm, boxes_hbm, imgw_hbm, imgh_hbm, out_hbm,
             win, cand_val, cand_idx, hist, btot, rcum,
             sval, sidx, boxbuf, imgw, imgh, outbuf):
    nc = 2
    b = lax.axis_index("s") * nc + lax.axis_index("c")
    lane = lax.iota(jnp.int32, LANE)

    pltpu.sync_copy(boxes_hbm.at[pl.ds(b * Q * 4, Q * 4)], boxbuf)
    pltpu.sync_copy(imgw_hbm.at[pl.ds(b * LANE, LANE)], imgw)
    pltpu.sync_copy(imgh_hbm.at[pl.ds(b * LANE, LANE)], imgh)

    def rebuild(n, _theta):
        return _histo_select(cand_val, cand_idx, hist, btot, rcum, n)

    def wbody(w, c):
        n, theta = c
        pltpu.sync_copy(p_hbm.at[pl.ds(b * QP * CP + w * WIN, WIN)], win)
        wbase = w * WIN

        def chunkbody(ch, c2):
            n2, th2 = c2
            n2, th2 = lax.cond(n2 >= REBUILD_AT, rebuild,
                               lambda a, t: (a, t), n2, th2)
            cb = ch * SUB
            mx = win[pl.ds(cb, LANE)]
            for i in range(1, SUB // LANE):
                mx = jnp.maximum(mx, win[pl.ds(cb + i * LANE, LANE)])
            hit = jnp.max(mx) > th2

            def phase2(nn):
                base = wbase + cb + lane
                for i in range(SUB // LANE):
                    v = win[pl.ds(cb + i * LANE, LANE)]
                    m = v > th2
                    plsc.store_compressed(cand_val.at[pl.ds(nn, LANE)],
                                          v, mask=m)
                    plsc.store_compressed(cand_idx.at[pl.ds(nn, LANE)],
                                          base + i * LANE, mask=m)
                    nn = nn + plsc.all_reduce_population_count(m)[0]
                return nn

            n2 = lax.cond(hit, phase2, lambda nn: nn, n2)
            return n2, th2

        return lax.fori_loop(0, WIN // SUB, chunkbody, (n, theta))

    n, _ = lax.fori_loop(0, NWIN, wbody,
                         (jnp.int32(0), jnp.float32(-0.5)))

    m, _ = _histo_select(cand_val, cand_idx, hist, btot, rcum, n)

    nv = (m + LANE - 1) // LANE

    def rankbody(iv, _):
        vi = cand_val[pl.ds(iv * LANE, LANE)]
        ii = cand_idx[pl.ds(iv * LANE, LANE)]
        lm = (iv * LANE + lane) < m

        def jbody(j, r):
            js = jnp.full((LANE,), j, jnp.int32)
            vj = plsc.load_gather(cand_val, [js])
            ij = plsc.load_gather(cand_idx, [js])
            beat = (vj > vi) | ((vj == vi) & (ij < ii))
            return r + beat.astype(jnp.int32)

        rank = lax.fori_loop(0, m, jbody, jnp.zeros((LANE,), jnp.int32))
        ok = lm & (rank < NUM_SELECT)
        plsc.store_scatter(sval, [rank], vi, mask=ok)
        plsc.store_scatter(sidx, [rank], ii, mask=ok)
        return 0

    lax.fori_loop(0, nv, rankbody, 0)

    wv = imgw[...]
    hv = imgh[...]
    inv5 = jnp.float32(0.2)
    for k in range(NUM_SELECT // LANE + 1):
        r = k * LANE + lane
        ok = r < NUM_SELECT
        v = sval[pl.ds(k * LANE, LANE)]
        ix = sidx[pl.ds(k * LANE, LANE)]
        t = lax.shift_right_logical(ix, 8)
        q = (t.astype(jnp.float32) * inv5).astype(jnp.int32)
        label = ix - q * CP
        bq = q * 4
        cx = plsc.load_gather(boxbuf, [bq], mask=ok)
        cy = plsc.load_gather(boxbuf, [bq + 1], mask=ok)
        bw = plsc.load_gather(boxbuf, [bq + 2], mask=ok)
        bh = plsc.load_gather(boxbuf, [bq + 3], mask=ok)
        x0 = (cx - 0.5 * bw) * wv
        y0 = (cy - 0.5 * bh) * hv
        x1 = (cx + 0.5 * bw) * wv
        y1 = (cy + 0.5 * bh) * hv
        base6 = r * 6
        plsc.store_scatter(outbuf, [base6], x0, mask=ok)
        plsc.store_scatter(outbuf, [base6 + 1], y0, mask=ok)
        plsc.store_scatter(outbuf, [base6 + 2], x1, mask=ok)
        plsc.store_scatter(outbuf, [base6 + 3], y1, mask=ok)
        plsc.store_scatter(outbuf, [base6 + 4], v, mask=ok)
        plsc.store_scatter(outbuf, [base6 + 5],
                           label.astype(jnp.float32), mask=ok)

    pltpu.sync_copy(outbuf, out_hbm.at[pl.ds(b * OUTW, OUTW)])


def _sc_select(p3, boxes, imgw, imgh):
    mesh = plsc.VectorSubcoreMesh(core_axis_name="c", subcore_axis_name="s")
    f = pl.kernel(
        _sc_body,
        out_type=jax.ShapeDtypeStruct((B * OUTW,), jnp.float32),
        mesh=mesh,
        compiler_params=pltpu.CompilerParams(needs_layout_passes=False),
        scratch_types=[
            pltpu.VMEM((WIN,), jnp.float32),
            pltpu.VMEM((CAP,), jnp.float32),
            pltpu.VMEM((CAP,), jnp.int32),
            pltpu.VMEM((NBINS * LANE,), jnp.int32),
            pltpu.VMEM((NBINS,), jnp.int32),
            pltpu.VMEM((NBINS,), jnp.int32),
            pltpu.VMEM((304,), jnp.float32),
            pltpu.VMEM((304,), jnp.int32),
            pltpu.VMEM((Q * 4,), jnp.float32),
            pltpu.VMEM((LANE,), jnp.float32),
            pltpu.VMEM((LANE,), jnp.float32),
            pltpu.VMEM((OUTW,), jnp.float32),
        ],
    )
    return f(p3, boxes, imgw, imgh)


def kernel(pred_logits, pred_boxes, pos_maps, target_sizes):
    p3 = _prob_to_label(pred_logits, pos_maps)
    img_h = target_sizes[:, 0].astype(jnp.float32)
    img_w = target_sizes[:, 1].astype(jnp.float32)
    imgw = jnp.broadcast_to(img_w[:, None], (B, LANE))
    imgh = jnp.broadcast_to(img_h[:, None], (B, LANE))
    out = _sc_select(p3.reshape(-1), pred_boxes.reshape(B * Q * 4),
                     imgw.reshape(-1), imgh.reshape(-1))
    return out.reshape(B, OUTW)[:, :NUM_SELECT * 6].reshape(
        B, NUM_SELECT, 6)


# speedup vs baseline: 34.7771x; 1.2001x over previous
import functools

import jax
import jax.numpy as jnp
from jax import lax
from jax.experimental import pallas as pl
from jax.experimental.pallas import tpu as pltpu
from jax.experimental.pallas import tpu_sc as plsc

NUM_SELECT = 300
CP = 1280
C_REAL = 1203
Q = 900
B = 32
T = 512
WIN = 38400
NWIN = Q * CP // WIN
SUB = 512
CAP = 4096
REBUILD_AT = 1024
NBINS = 512
LANE = 16
OUTW = 1824


def _matmul_body(l_ref, m_ref, out_ref):
    p = jax.nn.sigmoid(l_ref[...])
    acc = jnp.dot(p, m_ref[...], preferred_element_type=jnp.float32)
    col = lax.broadcasted_iota(jnp.int32, acc.shape, 1)
    out_ref[...] = jnp.where(col >= C_REAL, -1.0, acc)


def _prob_to_label(pred_logits, pos_maps):
    rows = B * Q
    RB = 384
    l2 = pred_logits.reshape(rows, T)
    mt = jnp.zeros((T, CP), jnp.float32).at[:, :C_REAL].set(pos_maps.T)
    out = pl.pallas_call(
        _matmul_body,
        grid=(rows // RB,),
        in_specs=[
            pl.BlockSpec((RB, T), lambda i: (i, 0)),
            pl.BlockSpec((T, CP), lambda i: (0, 0)),
        ],
        out_specs=pl.BlockSpec((RB, CP), lambda i: (i, 0)),
        out_shape=jax.ShapeDtypeStruct((rows, CP), jnp.float32),
    )(l2, mt)
    return out.reshape(B, Q, CP)


def _histo_select(cand_val, cand_idx, hist, btot, rcum, n):
    lane = lax.iota(jnp.int32, LANE)
    nv = (n + LANE - 1) // LANE
    inf16 = jnp.full((LANE,), jnp.inf, jnp.float32)
    ninf16 = jnp.full((LANE,), -jnp.inf, jnp.float32)

    def mm_body(i, c):
        mn, mx = c
        v = cand_val[pl.ds(i * LANE, LANE)]
        lm = (i * LANE + lane) < n
        mn = jnp.minimum(mn, jnp.where(lm, v, inf16))
        mx = jnp.maximum(mx, jnp.where(lm, v, ninf16))
        return mn, mx

    mnv, mxv = lax.fori_loop(0, nv, mm_body, (inf16, ninf16))
    lo = -jnp.max(-mnv)
    hi = jnp.max(mxv)
    rng = jnp.maximum(hi - lo, jnp.float32(0.0))
    e = lax.shift_right_logical(lax.bitcast_convert_type(rng, jnp.int32), 23)
    s = lax.bitcast_convert_type(lax.shift_left(253 - e, 23), jnp.float32)
    scale = s * NBINS

    def binof(v):
        b = ((v - lo) * scale).astype(jnp.int32)
        return jnp.clip(b, 0, NBINS - 1)

    zero16 = jnp.zeros((LANE,), jnp.int32)

    def zbody(g, _):
        hist[pl.ds(g * LANE, LANE)] = zero16
        return 0

    lax.fori_loop(0, NBINS, zbody, 0)

    ones16 = jnp.ones((LANE,), jnp.int32)

    def hbody(i, _):
        v = cand_val[pl.ds(i * LANE, LANE)]
        lm = (i * LANE + lane) < n
        b = binof(v)
        plsc.addupdate_scatter(hist, [b * LANE + lane], ones16, mask=lm)
        return 0

    lax.fori_loop(0, nv, hbody, 0)

    def tbody(g, _):
        base = (g * LANE + lane) * LANE
        acc = zero16
        for l in range(LANE):
            acc = acc + plsc.load_gather(hist, [base + l])
        btot[pl.ds(g * LANE, LANE)] = acc
        return 0

    lax.fori_loop(0, NBINS // LANE, tbody, 0)

    def rbody(g, c):
        carry, cnt = c
        gg = (NBINS // LANE - 1) - g
        ch = btot[pl.ds(gg * LANE, LANE)]
        cs = plsc.cumsum(lax.rev(ch, (0,))) + carry
        rc = lax.rev(cs, (0,))
        rcum[pl.ds(gg * LANE, LANE)] = rc
        cnt = cnt + jnp.sum((rc >= NUM_SELECT).astype(jnp.int32))
        return jnp.max(cs), cnt

    _, cnt = lax.fori_loop(0, NBINS // LANE, rbody,
                           (jnp.int32(0), jnp.int32(0)))
    istar = cnt - 1

    def cbody(i, c):
        k, mn = c
        v = cand_val[pl.ds(i * LANE, LANE)]
        ix = cand_idx[pl.ds(i * LANE, LANE)]
        lm = (i * LANE + lane) < n
        keep = lm & (binof(v) >= istar)
        cs = plsc.cumsum(keep.astype(jnp.int32))
        pos = k + cs - 1
        plsc.store_scatter(cand_val, [pos], v, mask=keep)
        plsc.store_scatter(cand_idx, [pos], ix, mask=keep)
        mn = jnp.minimum(mn, jnp.where(keep, v, inf16))
        return k + jnp.max(cs), mn

    k, mnv2 = lax.fori_loop(0, nv, cbody, (jnp.int32(0), inf16))
    return k, -jnp.max(-mnv2)


def _sc_body(p_hbm, boxes_hbm, imgw_hbm, imgh_hbm, out_hbm,
             win, cand_val, cand_idx, hist, btot, rcum,
             sval, sidx, boxbuf, imgw, imgh, outbuf):
    nc = 2
    b = lax.axis_index("s") * nc + lax.axis_index("c")
    lane = lax.iota(jnp.int32, LANE)

    pltpu.sync_copy(boxes_hbm.at[pl.ds(b * Q * 4, Q * 4)], boxbuf)
    pltpu.sync_copy(imgw_hbm.at[pl.ds(b * LANE, LANE)], imgw)
    pltpu.sync_copy(imgh_hbm.at[pl.ds(b * LANE, LANE)], imgh)

    def rebuild(n, _theta):
        return _histo_select(cand_val, cand_idx, hist, btot, rcum, n)

    def wbody(w, c):
        n, theta = c
        pltpu.sync_copy(p_hbm.at[pl.ds(b * Q * CP + w * WIN, WIN)], win)
        wbase = w * WIN

        def chunkbody(ch, c2):
            n2, th2 = c2
            n2, th2 = lax.cond(n2 >= REBUILD_AT, rebuild,
                               lambda a, t: (a, t), n2, th2)
            cb = ch * SUB
            mx = win[pl.ds(cb, LANE)]
            for i in range(1, SUB // LANE):
                mx = jnp.maximum(mx, win[pl.ds(cb + i * LANE, LANE)])
            hit = jnp.max(mx) > th2

            def phase2(nn):
                base = wbase + cb + lane
                for i in range(SUB // LANE):
                    v = win[pl.ds(cb + i * LANE, LANE)]
                    m = v > th2
                    plsc.store_compressed(cand_val.at[pl.ds(nn, LANE)],
                                          v, mask=m)
                    plsc.store_compressed(cand_idx.at[pl.ds(nn, LANE)],
                                          base + i * LANE, mask=m)
                    nn = nn + plsc.all_reduce_population_count(m)[0]
                return nn

            n2 = lax.cond(hit, phase2, lambda nn: nn, n2)
            return n2, th2

        return lax.fori_loop(0, WIN // SUB, chunkbody, (n, theta))

    n, _ = lax.fori_loop(0, NWIN, wbody,
                         (jnp.int32(0), jnp.float32(-0.5)))

    m, _ = _histo_select(cand_val, cand_idx, hist, btot, rcum, n)

    nv = (m + LANE - 1) // LANE

    def rankbody(iv, _):
        vi = cand_val[pl.ds(iv * LANE, LANE)]
        ii = cand_idx[pl.ds(iv * LANE, LANE)]
        lm = (iv * LANE + lane) < m

        def jbody(j, r):
            js = jnp.full((LANE,), j, jnp.int32)
            vj = plsc.load_gather(cand_val, [js])
            ij = plsc.load_gather(cand_idx, [js])
            beat = (vj > vi) | ((vj == vi) & (ij < ii))
            return r + beat.astype(jnp.int32)

        rank = lax.fori_loop(0, m, jbody, jnp.zeros((LANE,), jnp.int32))
        ok = lm & (rank < NUM_SELECT)
        plsc.store_scatter(sval, [rank], vi, mask=ok)
        plsc.store_scatter(sidx, [rank], ii, mask=ok)
        return 0

    lax.fori_loop(0, nv, rankbody, 0)

    wv = imgw[...]
    hv = imgh[...]
    inv5 = jnp.float32(0.2)
    for k in range(NUM_SELECT // LANE + 1):
        r = k * LANE + lane
        ok = r < NUM_SELECT
        v = sval[pl.ds(k * LANE, LANE)]
        ix = sidx[pl.ds(k * LANE, LANE)]
        t = lax.shift_right_logical(ix, 8)
        q = (t.astype(jnp.float32) * inv5).astype(jnp.int32)
        label = ix - q * CP
        bq = q * 4
        cx = plsc.load_gather(boxbuf, [bq], mask=ok)
        cy = plsc.load_gather(boxbuf, [bq + 1], mask=ok)
        bw = plsc.load_gather(boxbuf, [bq + 2], mask=ok)
        bh = plsc.load_gather(boxbuf, [bq + 3], mask=ok)
        x0 = (cx - 0.5 * bw) * wv
        y0 = (cy - 0.5 * bh) * hv
        x1 = (cx + 0.5 * bw) * wv
        y1 = (cy + 0.5 * bh) * hv
        base6 = r * 6
        plsc.store_scatter(outbuf, [base6], x0, mask=ok)
        plsc.store_scatter(outbuf, [base6 + 1], y0, mask=ok)
        plsc.store_scatter(outbuf, [base6 + 2], x1, mask=ok)
        plsc.store_scatter(outbuf, [base6 + 3], y1, mask=ok)
        plsc.store_scatter(outbuf, [base6 + 4], v, mask=ok)
        plsc.store_scatter(outbuf, [base6 + 5],
                           label.astype(jnp.float32), mask=ok)

    pltpu.sync_copy(outbuf, out_hbm.at[pl.ds(b * OUTW, OUTW)])


def _sc_select(p3, boxes, imgw, imgh):
    mesh = plsc.VectorSubcoreMesh(core_axis_name="c", subcore_axis_name="s")
    f = pl.kernel(
        _sc_body,
        out_type=jax.ShapeDtypeStruct((B * OUTW,), jnp.float32),
        mesh=mesh,
        compiler_params=pltpu.CompilerParams(needs_layout_passes=False),
        scratch_types=[
            pltpu.VMEM((WIN,), jnp.float32),
            pltpu.VMEM((CAP,), jnp.float32),
            pltpu.VMEM((CAP,), jnp.int32),
            pltpu.VMEM((NBINS * LANE,), jnp.int32),
            pltpu.VMEM((NBINS,), jnp.int32),
            pltpu.VMEM((NBINS,), jnp.int32),
            pltpu.VMEM((304,), jnp.float32),
            pltpu.VMEM((304,), jnp.int32),
            pltpu.VMEM((Q * 4,), jnp.float32),
            pltpu.VMEM((LANE,), jnp.float32),
            pltpu.VMEM((LANE,), jnp.float32),
            pltpu.VMEM((OUTW,), jnp.float32),
        ],
    )
    return f(p3, boxes, imgw, imgh)


def kernel(pred_logits, pred_boxes, pos_maps, target_sizes):
    p3 = _prob_to_label(pred_logits, pos_maps)
    img_h = target_sizes[:, 0].astype(jnp.float32)
    img_w = target_sizes[:, 1].astype(jnp.float32)
    imgw = jnp.broadcast_to(img_w[:, None], (B, LANE))
    imgh = jnp.broadcast_to(img_h[:, None], (B, LANE))
    out = _sc_select(p3.reshape(-1), pred_boxes.reshape(B * Q * 4),
                     imgw.reshape(-1), imgh.reshape(-1))
    return out.reshape(B, OUTW)[:, :NUM_SELECT * 6].reshape(
        B, NUM_SELECT, 6)


# speedup vs baseline: 37.3782x; 1.0748x over previous
import functools

import jax
import jax.numpy as jnp
from jax import lax
from jax.experimental import pallas as pl
from jax.experimental.pallas import tpu as pltpu
from jax.experimental.pallas import tpu_sc as plsc

NUM_SELECT = 300
CP = 1280
C_REAL = 1203
Q = 900
B = 32
T = 512
WIN = 38400
NWIN = Q * CP // WIN
SUB = 512
CAP = 4096
REBUILD_AT = 1024
NBINS = 512
LANE = 16
OUTW = 1824


def _matmul_body(l_ref, m_ref, out_ref):
    p = jax.nn.sigmoid(l_ref[...])
    acc = jnp.dot(p, m_ref[...], preferred_element_type=jnp.float32)
    col = lax.broadcasted_iota(jnp.int32, acc.shape, 1)
    out_ref[...] = jnp.where(col >= C_REAL, -1.0, acc)


def _prob_to_label(pred_logits, pos_maps):
    rows = B * Q
    RB = 384
    l2 = pred_logits.reshape(rows, T)
    mt = jnp.zeros((T, CP), jnp.float32).at[:, :C_REAL].set(pos_maps.T)
    out = pl.pallas_call(
        _matmul_body,
        grid=(rows // RB,),
        in_specs=[
            pl.BlockSpec((RB, T), lambda i: (i, 0)),
            pl.BlockSpec((T, CP), lambda i: (0, 0)),
        ],
        out_specs=pl.BlockSpec((RB, CP), lambda i: (i, 0)),
        out_shape=jax.ShapeDtypeStruct((rows, CP), jnp.float32),
    )(l2, mt)
    return out.reshape(B, Q, CP)


def _histo_select(cand_val, cand_idx, hist, btot, rcum, n):
    lane = lax.iota(jnp.int32, LANE)
    nv = (n + LANE - 1) // LANE
    inf16 = jnp.full((LANE,), jnp.inf, jnp.float32)
    ninf16 = jnp.full((LANE,), -jnp.inf, jnp.float32)

    def mm_body(i, c):
        mn, mx = c
        v = cand_val[pl.ds(i * LANE, LANE)]
        lm = (i * LANE + lane) < n
        mn = jnp.minimum(mn, jnp.where(lm, v, inf16))
        mx = jnp.maximum(mx, jnp.where(lm, v, ninf16))
        return mn, mx

    mnv, mxv = lax.fori_loop(0, nv, mm_body, (inf16, ninf16))
    lo = -jnp.max(-mnv)
    hi = jnp.max(mxv)
    rng = jnp.maximum(hi - lo, jnp.float32(0.0))
    e = lax.shift_right_logical(lax.bitcast_convert_type(rng, jnp.int32), 23)
    s = lax.bitcast_convert_type(lax.shift_left(253 - e, 23), jnp.float32)
    scale = s * NBINS

    def binof(v):
        b = ((v - lo) * scale).astype(jnp.int32)
        return jnp.clip(b, 0, NBINS - 1)

    zero16 = jnp.zeros((LANE,), jnp.int32)

    def zbody(g, _):
        hist[pl.ds(g * LANE, LANE)] = zero16
        return 0

    lax.fori_loop(0, NBINS, zbody, 0)

    ones16 = jnp.ones((LANE,), jnp.int32)

    def hbody(i, _):
        v = cand_val[pl.ds(i * LANE, LANE)]
        lm = (i * LANE + lane) < n
        b = binof(v)
        plsc.addupdate_scatter(hist, [b * LANE + lane], ones16, mask=lm)
        return 0

    lax.fori_loop(0, nv, hbody, 0)

    def tbody(g, _):
        base = (g * LANE + lane) * LANE
        acc = zero16
        for l in range(LANE):
            acc = acc + plsc.load_gather(hist, [base + l])
        btot[pl.ds(g * LANE, LANE)] = acc
        return 0

    lax.fori_loop(0, NBINS // LANE, tbody, 0)

    def rbody(g, c):
        carry, cnt = c
        gg = (NBINS // LANE - 1) - g
        ch = btot[pl.ds(gg * LANE, LANE)]
        cs = plsc.cumsum(lax.rev(ch, (0,))) + carry
        rc = lax.rev(cs, (0,))
        rcum[pl.ds(gg * LANE, LANE)] = rc
        cnt = cnt + jnp.sum((rc >= NUM_SELECT).astype(jnp.int32))
        return jnp.max(cs), cnt

    _, cnt = lax.fori_loop(0, NBINS // LANE, rbody,
                           (jnp.int32(0), jnp.int32(0)))
    istar = cnt - 1

    def cbody(i, c):
        k, mn = c
        v = cand_val[pl.ds(i * LANE, LANE)]
        ix = cand_idx[pl.ds(i * LANE, LANE)]
        lm = (i * LANE + lane) < n
        keep = lm & (binof(v) >= istar)
        cs = plsc.cumsum(keep.astype(jnp.int32))
        pos = k + cs - 1
        plsc.store_scatter(cand_val, [pos], v, mask=keep)
        plsc.store_scatter(cand_idx, [pos], ix, mask=keep)
        mn = jnp.minimum(mn, jnp.where(keep, v, inf16))
        return k + jnp.max(cs), mn

    k, mnv2 = lax.fori_loop(0, nv, cbody, (jnp.int32(0), inf16))
    return k, -jnp.max(-mnv2)


def _sc_body(p_hbm, boxes_hbm, imgw_hbm, imgh_hbm, out_hbm,
             win, win2, cand_val, cand_idx, hist, btot, rcum,
             sval, sidx, boxbuf, imgw, imgh, outbuf, sem0, sem1):
    nc = 2
    b = lax.axis_index("s") * nc + lax.axis_index("c")
    lane = lax.iota(jnp.int32, LANE)

    pltpu.sync_copy(boxes_hbm.at[pl.ds(b * Q * 4, Q * 4)], boxbuf)
    pltpu.sync_copy(imgw_hbm.at[pl.ds(b * LANE, LANE)], imgw)
    pltpu.sync_copy(imgh_hbm.at[pl.ds(b * LANE, LANE)], imgh)

    def rebuild(n, _theta):
        return _histo_select(cand_val, cand_idx, hist, btot, rcum, n)

    def start_win(w, buf, sem):
        pltpu.async_copy(p_hbm.at[pl.ds(b * Q * CP + w * WIN, WIN)], buf, sem)

    def wait_win(w, buf, sem):
        pltpu.make_async_copy(p_hbm.at[pl.ds(b * Q * CP + w * WIN, WIN)],
                              buf, sem).wait()

    def process(win, w, n, theta):
        wbase = w * WIN

        def chunkbody(ch, c2):
            n2, th2 = c2
            n2, th2 = lax.cond(n2 >= REBUILD_AT, rebuild,
                               lambda a, t: (a, t), n2, th2)
            cb = ch * SUB
            mx = win[pl.ds(cb, LANE)]
            for i in range(1, SUB // LANE):
                mx = jnp.maximum(mx, win[pl.ds(cb + i * LANE, LANE)])
            hit = jnp.max(mx) > th2

            def phase2(nn):
                base = wbase + cb + lane
                for i in range(SUB // LANE):
                    v = win[pl.ds(cb + i * LANE, LANE)]
                    m = v > th2
                    plsc.store_compressed(cand_val.at[pl.ds(nn, LANE)],
                                          v, mask=m)
                    plsc.store_compressed(cand_idx.at[pl.ds(nn, LANE)],
                                          base + i * LANE, mask=m)
                    nn = nn + plsc.all_reduce_population_count(m)[0]
                return nn

            n2 = lax.cond(hit, phase2, lambda nn: nn, n2)
            return n2, th2

        return lax.fori_loop(0, WIN // SUB, chunkbody, (n, theta))

    start_win(0, win, sem0)

    def wbody(t, c):
        n, theta = c
        w0 = 2 * t
        lax.cond(w0 + 1 < NWIN,
                 lambda: start_win(w0 + 1, win2, sem1), lambda: None)
        wait_win(w0, win, sem0)
        n, theta = process(win, w0, n, theta)
        w1 = w0 + 1
        lax.cond(w1 + 1 < NWIN,
                 lambda: start_win(w1 + 1, win, sem0), lambda: None)
        wait_win(w1, win2, sem1)
        n, theta = process(win2, w1, n, theta)
        return n, theta

    n, _ = lax.fori_loop(0, NWIN // 2, wbody,
                         (jnp.int32(0), jnp.float32(-0.5)))

    m, _ = _histo_select(cand_val, cand_idx, hist, btot, rcum, n)

    nv = (m + LANE - 1) // LANE

    def rankbody(iv, _):
        vi = cand_val[pl.ds(iv * LANE, LANE)]
        ii = cand_idx[pl.ds(iv * LANE, LANE)]
        lm = (iv * LANE + lane) < m

        def jbody(j, r):
            js = jnp.full((LANE,), j, jnp.int32)
            vj = plsc.load_gather(cand_val, [js])
            ij = plsc.load_gather(cand_idx, [js])
            beat = (vj > vi) | ((vj == vi) & (ij < ii))
            return r + beat.astype(jnp.int32)

        rank = lax.fori_loop(0, m, jbody, jnp.zeros((LANE,), jnp.int32))
        ok = lm & (rank < NUM_SELECT)
        plsc.store_scatter(sval, [rank], vi, mask=ok)
        plsc.store_scatter(sidx, [rank], ii, mask=ok)
        return 0

    lax.fori_loop(0, nv, rankbody, 0)

    wv = imgw[...]
    hv = imgh[...]
    inv5 = jnp.float32(0.2)
    for k in range(NUM_SELECT // LANE + 1):
        r = k * LANE + lane
        ok = r < NUM_SELECT
        v = sval[pl.ds(k * LANE, LANE)]
        ix = sidx[pl.ds(k * LANE, LANE)]
        t = lax.shift_right_logical(ix, 8)
        q = (t.astype(jnp.float32) * inv5).astype(jnp.int32)
        label = ix - q * CP
        bq = q * 4
        cx = plsc.load_gather(boxbuf, [bq], mask=ok)
        cy = plsc.load_gather(boxbuf, [bq + 1], mask=ok)
        bw = plsc.load_gather(boxbuf, [bq + 2], mask=ok)
        bh = plsc.load_gather(boxbuf, [bq + 3], mask=ok)
        x0 = (cx - 0.5 * bw) * wv
        y0 = (cy - 0.5 * bh) * hv
        x1 = (cx + 0.5 * bw) * wv
        y1 = (cy + 0.5 * bh) * hv
        base6 = r * 6
        plsc.store_scatter(outbuf, [base6], x0, mask=ok)
        plsc.store_scatter(outbuf, [base6 + 1], y0, mask=ok)
        plsc.store_scatter(outbuf, [base6 + 2], x1, mask=ok)
        plsc.store_scatter(outbuf, [base6 + 3], y1, mask=ok)
        plsc.store_scatter(outbuf, [base6 + 4], v, mask=ok)
        plsc.store_scatter(outbuf, [base6 + 5],
                           label.astype(jnp.float32), mask=ok)

    pltpu.sync_copy(outbuf, out_hbm.at[pl.ds(b * OUTW, OUTW)])


def _sc_select(p3, boxes, imgw, imgh):
    mesh = plsc.VectorSubcoreMesh(core_axis_name="c", subcore_axis_name="s")
    f = pl.kernel(
        _sc_body,
        out_type=jax.ShapeDtypeStruct((B * OUTW,), jnp.float32),
        mesh=mesh,
        compiler_params=pltpu.CompilerParams(needs_layout_passes=False),
        scratch_types=[
            pltpu.VMEM((WIN,), jnp.float32),
            pltpu.VMEM((WIN,), jnp.float32),
            pltpu.VMEM((CAP,), jnp.float32),
            pltpu.VMEM((CAP,), jnp.int32),
            pltpu.VMEM((NBINS * LANE,), jnp.int32),
            pltpu.VMEM((NBINS,), jnp.int32),
            pltpu.VMEM((NBINS,), jnp.int32),
            pltpu.VMEM((304,), jnp.float32),
            pltpu.VMEM((304,), jnp.int32),
            pltpu.VMEM((Q * 4,), jnp.float32),
            pltpu.VMEM((LANE,), jnp.float32),
            pltpu.VMEM((LANE,), jnp.float32),
            pltpu.VMEM((OUTW,), jnp.float32),
            pltpu.SemaphoreType.DMA,
            pltpu.SemaphoreType.DMA,
        ],
    )
    return f(p3, boxes, imgw, imgh)


def kernel(pred_logits, pred_boxes, pos_maps, target_sizes):
    p3 = _prob_to_label(pred_logits, pos_maps)
    img_h = target_sizes[:, 0].astype(jnp.float32)
    img_w = target_sizes[:, 1].astype(jnp.float32)
    imgw = jnp.broadcast_to(img_w[:, None], (B, LANE))
    imgh = jnp.broadcast_to(img_h[:, None], (B, LANE))
    out = _sc_select(p3.reshape(-1), pred_boxes.reshape(B * Q * 4),
                     imgw.reshape(-1), imgh.reshape(-1))
    return out.reshape(B, OUTW)[:, :NUM_SELECT * 6].reshape(
        B, NUM_SELECT, 6)


# speedup vs baseline: 38.0138x; 1.0170x over previous
import functools

import jax
import jax.numpy as jnp
from jax import lax
from jax.experimental import pallas as pl
from jax.experimental.pallas import tpu as pltpu
from jax.experimental.pallas import tpu_sc as plsc

NUM_SELECT = 300
CP = 1280
C_REAL = 1203
Q = 900
B = 32
T = 512
WIN = 38400
NWIN = Q * CP // WIN
SUB = 512
CAP = 4096
REBUILD_AT = 1024
NBINS = 512
LANE = 16
OUTW = 1824


def _matmul_body(l_ref, m_ref, out_ref):
    p = jax.nn.sigmoid(l_ref[...])
    acc = jnp.dot(p, m_ref[...], preferred_element_type=jnp.float32)
    col = lax.broadcasted_iota(jnp.int32, acc.shape, 1)
    out_ref[...] = jnp.where(col >= C_REAL, -1.0, acc)


def _prob_to_label(pred_logits, pos_maps):
    rows = B * Q
    RB = 640
    l2 = pred_logits.reshape(rows, T)
    mt = jnp.zeros((T, CP), jnp.float32).at[:, :C_REAL].set(pos_maps.T)
    out = pl.pallas_call(
        _matmul_body,
        grid=(rows // RB,),
        in_specs=[
            pl.BlockSpec((RB, T), lambda i: (i, 0)),
            pl.BlockSpec((T, CP), lambda i: (0, 0)),
        ],
        out_specs=pl.BlockSpec((RB, CP), lambda i: (i, 0)),
        out_shape=jax.ShapeDtypeStruct((rows, CP), jnp.float32),
    )(l2, mt)
    return out.reshape(B, Q, CP)


def _histo_select(cand_val, cand_idx, hist, btot, rcum, n):
    lane = lax.iota(jnp.int32, LANE)
    nv = (n + LANE - 1) // LANE
    inf16 = jnp.full((LANE,), jnp.inf, jnp.float32)
    ninf16 = jnp.full((LANE,), -jnp.inf, jnp.float32)

    def mm_body(i, c):
        mn, mx = c
        v = cand_val[pl.ds(i * LANE, LANE)]
        lm = (i * LANE + lane) < n
        mn = jnp.minimum(mn, jnp.where(lm, v, inf16))
        mx = jnp.maximum(mx, jnp.where(lm, v, ninf16))
        return mn, mx

    mnv, mxv = lax.fori_loop(0, nv, mm_body, (inf16, ninf16))
    lo = -jnp.max(-mnv)
    hi = jnp.max(mxv)
    rng = jnp.maximum(hi - lo, jnp.float32(0.0))
    e = lax.shift_right_logical(lax.bitcast_convert_type(rng, jnp.int32), 23)
    s = lax.bitcast_convert_type(lax.shift_left(253 - e, 23), jnp.float32)
    scale = s * NBINS

    def binof(v):
        b = ((v - lo) * scale).astype(jnp.int32)
        return jnp.clip(b, 0, NBINS - 1)

    zero16 = jnp.zeros((LANE,), jnp.int32)

    def zbody(g, _):
        hist[pl.ds(g * LANE, LANE)] = zero16
        return 0

    lax.fori_loop(0, NBINS, zbody, 0)

    ones16 = jnp.ones((LANE,), jnp.int32)

    def hbody(i, _):
        v = cand_val[pl.ds(i * LANE, LANE)]
        lm = (i * LANE + lane) < n
        b = binof(v)
        plsc.addupdate_scatter(hist, [b * LANE + lane], ones16, mask=lm)
        return 0

    lax.fori_loop(0, nv, hbody, 0)

    def tbody(g, _):
        base = (g * LANE + lane) * LANE
        acc = zero16
        for l in range(LANE):
            acc = acc + plsc.load_gather(hist, [base + l])
        btot[pl.ds(g * LANE, LANE)] = acc
        return 0

    lax.fori_loop(0, NBINS // LANE, tbody, 0)

    def rbody(g, c):
        carry, cnt = c
        gg = (NBINS // LANE - 1) - g
        ch = btot[pl.ds(gg * LANE, LANE)]
        cs = plsc.cumsum(lax.rev(ch, (0,))) + carry
        rc = lax.rev(cs, (0,))
        rcum[pl.ds(gg * LANE, LANE)] = rc
        cnt = cnt + jnp.sum((rc >= NUM_SELECT).astype(jnp.int32))
        return jnp.max(cs), cnt

    _, cnt = lax.fori_loop(0, NBINS // LANE, rbody,
                           (jnp.int32(0), jnp.int32(0)))
    istar = cnt - 1

    def cbody(i, c):
        k, mn = c
        v = cand_val[pl.ds(i * LANE, LANE)]
        ix = cand_idx[pl.ds(i * LANE, LANE)]
        lm = (i * LANE + lane) < n
        keep = lm & (binof(v) >= istar)
        cs = plsc.cumsum(keep.astype(jnp.int32))
        pos = k + cs - 1
        plsc.store_scatter(cand_val, [pos], v, mask=keep)
        plsc.store_scatter(cand_idx, [pos], ix, mask=keep)
        mn = jnp.minimum(mn, jnp.where(keep, v, inf16))
        return k + jnp.max(cs), mn

    k, mnv2 = lax.fori_loop(0, nv, cbody, (jnp.int32(0), inf16))
    return k, -jnp.max(-mnv2)


def _sc_body(p_hbm, boxes_hbm, imgw_hbm, imgh_hbm, out_hbm,
             win, win2, cand_val, cand_idx, hist, btot, rcum,
             sval, sidx, boxbuf, imgw, imgh, outbuf, sem0, sem1):
    nc = 2
    b = lax.axis_index("s") * nc + lax.axis_index("c")
    lane = lax.iota(jnp.int32, LANE)

    pltpu.sync_copy(boxes_hbm.at[pl.ds(b * Q * 4, Q * 4)], boxbuf)
    pltpu.sync_copy(imgw_hbm.at[pl.ds(b * LANE, LANE)], imgw)
    pltpu.sync_copy(imgh_hbm.at[pl.ds(b * LANE, LANE)], imgh)

    def rebuild(n, _theta):
        return _histo_select(cand_val, cand_idx, hist, btot, rcum, n)

    def start_win(w, buf, sem):
        pltpu.async_copy(p_hbm.at[pl.ds(b * Q * CP + w * WIN, WIN)], buf, sem)

    def wait_win(w, buf, sem):
        pltpu.make_async_copy(p_hbm.at[pl.ds(b * Q * CP + w * WIN, WIN)],
                              buf, sem).wait()

    def process(win, w, n, theta):
        wbase = w * WIN

        def chunkbody(ch, c2):
            n2, th2 = c2
            n2, th2 = lax.cond(n2 >= REBUILD_AT, rebuild,
                               lambda a, t: (a, t), n2, th2)
            cb = ch * SUB
            mx = win[pl.ds(cb, LANE)]
            for i in range(1, SUB // LANE):
                mx = jnp.maximum(mx, win[pl.ds(cb + i * LANE, LANE)])
            hit = jnp.max(mx) > th2

            def phase2(nn):
                base = wbase + cb + lane
                for i in range(SUB // LANE):
                    v = win[pl.ds(cb + i * LANE, LANE)]
                    m = v > th2
                    plsc.store_compressed(cand_val.at[pl.ds(nn, LANE)],
                                          v, mask=m)
                    plsc.store_compressed(cand_idx.at[pl.ds(nn, LANE)],
                                          base + i * LANE, mask=m)
                    nn = nn + plsc.all_reduce_population_count(m)[0]
                return nn

            n2 = lax.cond(hit, phase2, lambda nn: nn, n2)
            return n2, th2

        return lax.fori_loop(0, WIN // SUB, chunkbody, (n, theta))

    start_win(0, win, sem0)

    def wbody(t, c):
        n, theta = c
        w0 = 2 * t
        lax.cond(w0 + 1 < NWIN,
                 lambda: start_win(w0 + 1, win2, sem1), lambda: None)
        wait_win(w0, win, sem0)
        n, theta = process(win, w0, n, theta)
        w1 = w0 + 1
        lax.cond(w1 + 1 < NWIN,
                 lambda: start_win(w1 + 1, win, sem0), lambda: None)
        wait_win(w1, win2, sem1)
        n, theta = process(win2, w1, n, theta)
        return n, theta

    n, _ = lax.fori_loop(0, NWIN // 2, wbody,
                         (jnp.int32(0), jnp.float32(-0.5)))

    m, _ = _histo_select(cand_val, cand_idx, hist, btot, rcum, n)
    m, _ = _histo_select(cand_val, cand_idx, hist, btot, rcum, m)

    nv = (m + LANE - 1) // LANE

    def rankbody(iv, _):
        vi = cand_val[pl.ds(iv * LANE, LANE)]
        ii = cand_idx[pl.ds(iv * LANE, LANE)]
        lm = (iv * LANE + lane) < m

        def jbody(j, r):
            js = jnp.full((LANE,), j, jnp.int32)
            vj = plsc.load_gather(cand_val, [js])
            ij = plsc.load_gather(cand_idx, [js])
            beat = (vj > vi) | ((vj == vi) & (ij < ii))
            return r + beat.astype(jnp.int32)

        rank = lax.fori_loop(0, m, jbody, jnp.zeros((LANE,), jnp.int32))
        ok = lm & (rank < NUM_SELECT)
        plsc.store_scatter(sval, [rank], vi, mask=ok)
        plsc.store_scatter(sidx, [rank], ii, mask=ok)
        return 0

    lax.fori_loop(0, nv, rankbody, 0)

    wv = imgw[...]
    hv = imgh[...]
    inv5 = jnp.float32(0.2)
    for k in range(NUM_SELECT // LANE + 1):
        r = k * LANE + lane
        ok = r < NUM_SELECT
        v = sval[pl.ds(k * LANE, LANE)]
        ix = sidx[pl.ds(k * LANE, LANE)]
        t = lax.shift_right_logical(ix, 8)
        q = (t.astype(jnp.float32) * inv5).astype(jnp.int32)
        label = ix - q * CP
        bq = q * 4
        cx = plsc.load_gather(boxbuf, [bq], mask=ok)
        cy = plsc.load_gather(boxbuf, [bq + 1], mask=ok)
        bw = plsc.load_gather(boxbuf, [bq + 2], mask=ok)
        bh = plsc.load_gather(boxbuf, [bq + 3], mask=ok)
        x0 = (cx - 0.5 * bw) * wv
        y0 = (cy - 0.5 * bh) * hv
        x1 = (cx + 0.5 * bw) * wv
        y1 = (cy + 0.5 * bh) * hv
        base6 = r * 6
        plsc.store_scatter(outbuf, [base6], x0, mask=ok)
        plsc.store_scatter(outbuf, [base6 + 1], y0, mask=ok)
        plsc.store_scatter(outbuf, [base6 + 2], x1, mask=ok)
        plsc.store_scatter(outbuf, [base6 + 3], y1, mask=ok)
        plsc.store_scatter(outbuf, [base6 + 4], v, mask=ok)
        plsc.store_scatter(outbuf, [base6 + 5],
                           label.astype(jnp.float32), mask=ok)

    pltpu.sync_copy(outbuf, out_hbm.at[pl.ds(b * OUTW, OUTW)])


def _sc_select(p3, boxes, imgw, imgh):
    mesh = plsc.VectorSubcoreMesh(core_axis_name="c", subcore_axis_name="s")
    f = pl.kernel(
        _sc_body,
        out_type=jax.ShapeDtypeStruct((B * OUTW,), jnp.float32),
        mesh=mesh,
        compiler_params=pltpu.CompilerParams(needs_layout_passes=False),
        scratch_types=[
            pltpu.VMEM((WIN,), jnp.float32),
            pltpu.VMEM((WIN,), jnp.float32),
            pltpu.VMEM((CAP,), jnp.float32),
            pltpu.VMEM((CAP,), jnp.int32),
            pltpu.VMEM((NBINS * LANE,), jnp.int32),
            pltpu.VMEM((NBINS,), jnp.int32),
            pltpu.VMEM((NBINS,), jnp.int32),
            pltpu.VMEM((304,), jnp.float32),
            pltpu.VMEM((304,), jnp.int32),
            pltpu.VMEM((Q * 4,), jnp.float32),
            pltpu.VMEM((LANE,), jnp.float32),
            pltpu.VMEM((LANE,), jnp.float32),
            pltpu.VMEM((OUTW,), jnp.float32),
            pltpu.SemaphoreType.DMA,
            pltpu.SemaphoreType.DMA,
        ],
    )
    return f(p3, boxes, imgw, imgh)


def kernel(pred_logits, pred_boxes, pos_maps, target_sizes):
    p3 = _prob_to_label(pred_logits, pos_maps)
    img_h = target_sizes[:, 0].astype(jnp.float32)
    img_w = target_sizes[:, 1].astype(jnp.float32)
    imgw = jnp.broadcast_to(img_w[:, None], (B, LANE))
    imgh = jnp.broadcast_to(img_h[:, None], (B, LANE))
    out = _sc_select(p3.reshape(-1), pred_boxes.reshape(B * Q * 4),
                     imgw.reshape(-1), imgh.reshape(-1))
    return out.reshape(B, OUTW)[:, :NUM_SELECT * 6].reshape(
        B, NUM_SELECT, 6)
